# baseline (speedup 1.0000x reference)
import functools

import jax
import jax.numpy as jnp
from jax import lax
from jax.experimental import pallas as pl
from jax.experimental.pallas import tpu as pltpu
from jax.experimental.pallas import tpu_sc as plsc

D = 32
B = 16384
BAG = 50
BAGP = 128
NC = 2
NS = 16
NW = NC * NS
BPW = B // NW
L = 16

_mesh = plsc.VectorSubcoreMesh(core_axis_name="c", subcore_axis_name="s")


@functools.partial(
    pl.kernel,
    mesh=_mesh,
    out_type=jax.ShapeDtypeStruct((B, D), jnp.float32),
    scratch_types=[
        pltpu.VMEM((BPW, BAGP), jnp.int32),
        pltpu.VMEM((BAG, BPW), jnp.int32),
        pltpu.VMEM((BPW, D), jnp.float32),
        pltpu.SemaphoreType.DMA,
    ],
    compiler_params=pltpu.CompilerParams(
        use_tc_tiling_on_sc=False, needs_layout_passes=False
    ),
)
def _bag(idx_hbm, w_hbm, out_hbm, idx_raw, idx_t, acc, sem):
    wid = lax.axis_index("s") * NC + lax.axis_index("c")
    idx_cp = pltpu.async_copy(idx_hbm.at[pl.ds(wid * BPW, BPW), :], idx_raw, sem)
    zero = jnp.zeros((L,), jnp.float32)

    def zbody(i, carry):
        acc[i, pl.ds(0, L)] = zero
        acc[i, pl.ds(L, L)] = zero
        return carry

    lax.fori_loop(0, BPW, zbody, 0)
    idx_cp.wait()

    lane = lax.iota(jnp.int32, L)
    descs = []
    for j in range(BAG):
        def tbody(cc, carry, j=j):
            rows = cc * L + lane
            cols = jnp.full((L,), j, jnp.int32)
            vals = plsc.load_gather(idx_raw, [rows, cols])
            idx_t[j, pl.ds(cc * L, L)] = vals
            return carry

        lax.fori_loop(0, BPW // L, tbody, 0)
        descs.append(pltpu.async_copy(w_hbm.at[idx_t.at[j]], acc, sem, add=True))
    for d in descs:
        d.wait()
    pltpu.sync_copy(acc, out_hbm.at[pl.ds(wid * BPW, BPW)])


_PAD_BLK = 2048


def _pad_body(x_ref, o_ref):
    o_ref[...] = jnp.concatenate(
        [x_ref[...], jnp.zeros((_PAD_BLK, BAGP - BAG), jnp.int32)], axis=1
    )


_pad = pl.pallas_call(
    _pad_body,
    out_shape=jax.ShapeDtypeStruct((B, BAGP), jnp.int32),
    grid=(B // _PAD_BLK,),
    in_specs=[pl.BlockSpec((_PAD_BLK, BAG), lambda i: (i, 0))],
    out_specs=pl.BlockSpec((_PAD_BLK, BAGP), lambda i: (i, 0)),
)


def kernel(input, weight):
    idx = _pad(input.astype(jnp.int32))
    return _bag(idx, weight)


# speedup vs baseline: 1.0246x; 1.0246x over previous
import functools

import jax
import jax.numpy as jnp
from jax import lax
from jax.experimental import pallas as pl
from jax.experimental.pallas import tpu as pltpu
from jax.experimental.pallas import tpu_sc as plsc

D = 32
B = 16384
BAG = 50
NC = 2
NS = 16
NW = NC * NS
BPW = B // NW
L = 16

_mesh = plsc.VectorSubcoreMesh(core_axis_name="c", subcore_axis_name="s")


@functools.partial(
    pl.kernel,
    mesh=_mesh,
    out_type=jax.ShapeDtypeStruct((B, D), jnp.float32),
    scratch_types=[
        pltpu.VMEM((BAG, BPW), jnp.int32),
        pltpu.VMEM((BPW, D), jnp.float32),
        pltpu.SemaphoreType.DMA,
    ],
    compiler_params=pltpu.CompilerParams(use_tc_tiling_on_sc=False),
)
def _bag(idx_hbm, w_hbm, out_hbm, idx_v, acc, sem):
    wid = lax.axis_index("s") * NC + lax.axis_index("c")
    idx_cp = pltpu.async_copy(idx_hbm.at[wid], idx_v, sem)
    zero = jnp.zeros((L,), jnp.float32)

    def zbody(i, carry):
        acc[i, pl.ds(0, L)] = zero
        acc[i, pl.ds(L, L)] = zero
        return carry

    lax.fori_loop(0, BPW, zbody, 0)
    idx_cp.wait()
    descs = [
        pltpu.async_copy(w_hbm.at[idx_v.at[j]], acc, sem, add=True)
        for j in range(BAG)
    ]
    for d in descs:
        d.wait()
    pltpu.sync_copy(acc, out_hbm.at[pl.ds(wid * BPW, BPW)])


def kernel(input, weight):
    idx = input.astype(jnp.int32)
    idx_r = idx.reshape(NW, BPW, BAG).transpose(0, 2, 1)
    return _bag(idx_r, weight)
